# baseline (speedup 1.0000x reference)
import functools

import jax
import jax.numpy as jnp
from jax import lax
from jax.experimental import pallas as pl
from jax.experimental.pallas import tpu as pltpu
from jax.experimental.pallas import tpu_sc as plsc

N = 10000
E = 160000
D = 256
ED = 16
L = 3

_BE = 2048
_BN = 512

_NSUB = 16
_CK = 128
_CHN = 80
_EPW = _CHN * _CK
_EPAD = _NSUB * _EPW
_RPT = 632
_NSP = _NSUB * _RPT
_RCOPIES = ((0, 128), (128, 128), (256, 128), (384, 128), (512, 120))


def _seg_body(hr, e3, src3, dst3, out, idx0, idx1, dc0, dc1, a_v, b0, b1,
              agg_sh, gsem, esem):
    c = lax.axis_index("c")
    s = lax.axis_index("s")
    idx = (idx0, idx1)
    dc = (dc0, dc1)
    b = (b0, b1)
    zero16 = jnp.zeros((16,), jnp.float32)

    def zrow(r, carry):
        for jj in range(8):
            a_v[r, pl.ds(jj * 16, 16)] = zero16
            b1[r, pl.ds(jj * 16, 16)] = zero16
        return carry

    lax.fori_loop(0, _CK, zrow, 0)
    for k in range(_CK // 16):
        dc1[pl.ds(k * 16, 16)] = jnp.full((16,), N, jnp.int32)

    base = s * _RPT
    for off, sz in _RCOPIES:
        pltpu.sync_copy(a_v.at[pl.ds(0, sz)], agg_sh.at[pl.ds(base + off, sz)])
    plsc.subcore_barrier()

    eb0 = s * _EPW

    def outer(jj, carry):
        for u in (0, 1):
            j = jj * 2 + u
            pltpu.sync_copy(src3.at[pl.ds(eb0 + j * _CK, _CK)], idx[u])
            pltpu.sync_copy(dst3.at[pl.ds(eb0 + j * _CK, _CK)], dc[u])
            for k in range(_CK // 16):
                v = idx[u][pl.ds(k * 16, 16)]
                idx[u][pl.ds(k * 16, 16)] = v * 2 + c
            g = pltpu.async_copy(hr.at[idx[u]], a_v, gsem)
            e = pltpu.async_copy(e3.at[c, pl.ds(eb0 + j * _CK, _CK)], b[u],
                                 esem)
            pltpu.sync_copy(b[1 - u], agg_sh.at[dc[1 - u]], add=True)
            g.wait()
            e.wait()

            def crow(rr, inner):
                for r2 in range(4):
                    for jj2 in range(8):
                        r = rr * 4 + r2
                        av = a_v[r, pl.ds(jj2 * 16, 16)]
                        bv = b[u][r, pl.ds(jj2 * 16, 16)]
                        b[u][r, pl.ds(jj2 * 16, 16)] = jnp.maximum(av + bv, 0.0)
                return inner

            lax.fori_loop(0, _CK // 4, crow, 0)
        return carry

    lax.fori_loop(0, _CHN // 2, outer, 0)
    pltpu.sync_copy(b1, agg_sh.at[dc1], add=True)
    plsc.subcore_barrier()

    for off, sz in _RCOPIES:
        pltpu.sync_copy(agg_sh.at[pl.ds(base + off, sz)],
                        out.at[c, pl.ds(base + off, sz)])


_seg_call = functools.partial(
    pl.kernel,
    _seg_body,
    out_type=jax.ShapeDtypeStruct((2, _NSP, 128), jnp.float32),
    mesh=plsc.VectorSubcoreMesh(core_axis_name="c", subcore_axis_name="s"),
    scratch_types=[
        pltpu.VMEM((_CK,), jnp.int32),
        pltpu.VMEM((_CK,), jnp.int32),
        pltpu.VMEM((_CK,), jnp.int32),
        pltpu.VMEM((_CK,), jnp.int32),
        pltpu.VMEM((_CK, 128), jnp.float32),
        pltpu.VMEM((_CK, 128), jnp.float32),
        pltpu.VMEM((_CK, 128), jnp.float32),
        pltpu.VMEM_SHARED((_NSP, 128), jnp.float32),
        pltpu.SemaphoreType.DMA,
        pltpu.SemaphoreType.DMA,
    ],
)()


def _segment(h, e3, src3, dst3):
    hr = h.reshape(2 * N, 128)
    return _seg_call(hr, e3, src3, dst3)


def _node_update_body(h_ref, agg_ref, w_ref, b_ref, g_ref, bt_ref, o_ref):
    agg = jnp.concatenate([agg_ref[0], agg_ref[1]], axis=-1)
    t = h_ref[...] + agg
    t = jnp.dot(t, w_ref[...], preferred_element_type=jnp.float32) + b_ref[...]
    mu = jnp.mean(t, axis=-1, keepdims=True)
    var = jnp.mean((t - mu) ** 2, axis=-1, keepdims=True)
    t = (t - mu) * jax.lax.rsqrt(var + 1e-5) * g_ref[...] + bt_ref[...]
    o_ref[...] = jnp.maximum(t, 0.0)


def _node_update(h, agg, w1, b1, g, bt):
    grid = (pl.cdiv(N, _BN),)
    return pl.pallas_call(
        _node_update_body,
        grid=grid,
        in_specs=[
            pl.BlockSpec((_BN, D), lambda i: (i, 0)),
            pl.BlockSpec((2, _BN, 128), lambda i: (0, i, 0)),
            pl.BlockSpec((D, D), lambda i: (0, 0)),
            pl.BlockSpec((1, D), lambda i: (0, 0)),
            pl.BlockSpec((1, D), lambda i: (0, 0)),
            pl.BlockSpec((1, D), lambda i: (0, 0)),
        ],
        out_specs=pl.BlockSpec((_BN, D), lambda i: (i, 0)),
        out_shape=jax.ShapeDtypeStruct((N, D), jnp.float32),
    )(h, agg, w1, b1.reshape(1, D), g.reshape(1, D), bt.reshape(1, D))


def _head_body(h_ref, w1_ref, b1_ref, w2_ref, b2_ref, o_ref):
    t = jnp.dot(h_ref[...], w1_ref[...], preferred_element_type=jnp.float32)
    t = jnp.maximum(t + b1_ref[...], 0.0)
    o_ref[...] = jnp.dot(t, w2_ref[...], preferred_element_type=jnp.float32) + b2_ref[...]


def _head(h, wh1, bh1, wh2, bh2):
    grid = (pl.cdiv(N, _BN),)
    return pl.pallas_call(
        _head_body,
        grid=grid,
        in_specs=[
            pl.BlockSpec((_BN, D), lambda i: (i, 0)),
            pl.BlockSpec((D, D), lambda i: (0, 0)),
            pl.BlockSpec((1, D), lambda i: (0, 0)),
            pl.BlockSpec((D, 1), lambda i: (0, 0)),
            pl.BlockSpec((1, 1), lambda i: (0, 0)),
        ],
        out_specs=pl.BlockSpec((_BN, 1), lambda i: (i, 0)),
        out_shape=jax.ShapeDtypeStruct((N, 1), jnp.float32),
    )(h, wh1, bh1.reshape(1, D), wh2, bh2.reshape(1, 1))


def _layernorm(t, g, bt):
    mu = jnp.mean(t, axis=-1, keepdims=True)
    var = jnp.mean((t - mu) ** 2, axis=-1, keepdims=True)
    return (t - mu) * jax.lax.rsqrt(var + 1e-5) * g + bt


def kernel(x, edge_index, edge_attr, params):
    src = edge_index[0].astype(jnp.int32)
    dst = edge_index[1].astype(jnp.int32)
    src3 = jnp.pad(src, (0, _EPAD - E))
    dst3 = jnp.pad(dst, (0, _EPAD - E), constant_values=N)

    ea_p = jnp.pad(edge_attr, ((0, _EPAD - E), (0, 0)))
    w_split = jnp.stack([params[f"We{i}"].reshape(ED, 2, 128).transpose(1, 0, 2)
                         for i in range(L)])
    b_split = jnp.stack([params[f"be{i}"].reshape(2, 1, 128) for i in range(L)])
    e_list = [jnp.einsum("ek,ckd->ced", ea_p, w_split[i],
                         preferred_element_type=jnp.float32) + b_split[i]
              for i in range(L)]

    h = x
    for i in range(L - 1):
        agg3 = _segment(h, e_list[i], src3, dst3)
        agg = jnp.concatenate([agg3[0, :N], agg3[1, :N]], axis=-1)
        t = (h + agg) @ params[f"W1{i}"] + params[f"b1{i}"]
        h = jnp.maximum(_layernorm(t, params[f"g{i}"], params[f"bt{i}"]), 0.0)

    i = L - 1
    agg3 = _segment(h, e_list[i], src3, dst3)
    h = _node_update(h, agg3, params[f"W1{i}"], params[f"b1{i}"],
                     params[f"g{i}"], params[f"bt{i}"])
    return _head(h, params["Wh1"], params["bh1"], params["Wh2"], params["bh2"])


# speedup vs baseline: 1.0025x; 1.0025x over previous
import functools

import jax
import jax.numpy as jnp
from jax import lax
from jax.experimental import pallas as pl
from jax.experimental.pallas import tpu as pltpu
from jax.experimental.pallas import tpu_sc as plsc

N = 10000
E = 160000
D = 256
ED = 16
L = 3

_BE = 2048
_BN = 512

_NSUB = 16
_CK = 128
_CHN = 80
_EPW = _CHN * _CK
_EPAD = _NSUB * _EPW
_RPT = 632
_NSP = _NSUB * _RPT
_RCOPIES = ((0, 128), (128, 128), (256, 128), (384, 128), (512, 120))


def _seg_body(hr, e3, src3, dst3, out, idx0, idx1, dc0, dc1, a_v, b0, b1,
              agg_sh, gsem, esem):
    c = lax.axis_index("c")
    s = lax.axis_index("s")
    idx = (idx0, idx1)
    dc = (dc0, dc1)
    b = (b0, b1)
    zero16 = jnp.zeros((16,), jnp.float32)

    def zrow(r, carry):
        for jj in range(8):
            a_v[r, pl.ds(jj * 16, 16)] = zero16
            b1[r, pl.ds(jj * 16, 16)] = zero16
        return carry

    lax.fori_loop(0, _CK, zrow, 0)
    for k in range(_CK // 16):
        dc1[pl.ds(k * 16, 16)] = jnp.full((16,), N, jnp.int32)

    base = s * _RPT
    for off, sz in _RCOPIES:
        pltpu.sync_copy(a_v.at[pl.ds(0, sz)], agg_sh.at[pl.ds(base + off, sz)])
    plsc.subcore_barrier()

    eb0 = s * _EPW

    def outer(jj, carry):
        for u in (0, 1):
            j = jj * 2 + u
            pltpu.sync_copy(src3.at[pl.ds(eb0 + j * _CK, _CK)], idx[u])
            pltpu.sync_copy(dst3.at[pl.ds(eb0 + j * _CK, _CK)], dc[u])
            for k in range(_CK // 16):
                v = idx[u][pl.ds(k * 16, 16)]
                idx[u][pl.ds(k * 16, 16)] = v * 2 + c
            g = pltpu.async_copy(hr.at[idx[u]], a_v, gsem)
            e = pltpu.async_copy(e3.at[c, pl.ds(eb0 + j * _CK, _CK)], b[u],
                                 esem)
            pltpu.sync_copy(b[1 - u], agg_sh.at[dc[1 - u]], add=True)
            g.wait()
            e.wait()

            def crow(r, inner):
                for jj2 in range(8):
                    av = a_v[r, pl.ds(jj2 * 16, 16)]
                    bv = b[u][r, pl.ds(jj2 * 16, 16)]
                    b[u][r, pl.ds(jj2 * 16, 16)] = jnp.maximum(av + bv, 0.0)
                return inner

            lax.fori_loop(0, _CK, crow, 0)
        return carry

    lax.fori_loop(0, _CHN // 2, outer, 0)
    pltpu.sync_copy(b1, agg_sh.at[dc1], add=True)
    plsc.subcore_barrier()

    for off, sz in _RCOPIES:
        pltpu.sync_copy(agg_sh.at[pl.ds(base + off, sz)],
                        out.at[c, pl.ds(base + off, sz)])


_seg_call = functools.partial(
    pl.kernel,
    _seg_body,
    out_type=jax.ShapeDtypeStruct((2, _NSP, 128), jnp.float32),
    mesh=plsc.VectorSubcoreMesh(core_axis_name="c", subcore_axis_name="s"),
    scratch_types=[
        pltpu.VMEM((_CK,), jnp.int32),
        pltpu.VMEM((_CK,), jnp.int32),
        pltpu.VMEM((_CK,), jnp.int32),
        pltpu.VMEM((_CK,), jnp.int32),
        pltpu.VMEM((_CK, 128), jnp.float32),
        pltpu.VMEM((_CK, 128), jnp.float32),
        pltpu.VMEM((_CK, 128), jnp.float32),
        pltpu.VMEM_SHARED((_NSP, 128), jnp.float32),
        pltpu.SemaphoreType.DMA,
        pltpu.SemaphoreType.DMA,
    ],
)()


def _segment(h, e3, src3, dst3):
    hr = h.reshape(2 * N, 128)
    return _seg_call(hr, e3, src3, dst3)


def _node_update_body(h_ref, agg_ref, w_ref, b_ref, g_ref, bt_ref, o_ref):
    agg = jnp.concatenate([agg_ref[0], agg_ref[1]], axis=-1)
    t = h_ref[...] + agg
    t = jnp.dot(t, w_ref[...], preferred_element_type=jnp.float32) + b_ref[...]
    mu = jnp.mean(t, axis=-1, keepdims=True)
    var = jnp.mean((t - mu) ** 2, axis=-1, keepdims=True)
    t = (t - mu) * jax.lax.rsqrt(var + 1e-5) * g_ref[...] + bt_ref[...]
    o_ref[...] = jnp.maximum(t, 0.0)


def _node_update(h, agg, w1, b1, g, bt):
    grid = (pl.cdiv(N, _BN),)
    return pl.pallas_call(
        _node_update_body,
        grid=grid,
        in_specs=[
            pl.BlockSpec((_BN, D), lambda i: (i, 0)),
            pl.BlockSpec((2, _BN, 128), lambda i: (0, i, 0)),
            pl.BlockSpec((D, D), lambda i: (0, 0)),
            pl.BlockSpec((1, D), lambda i: (0, 0)),
            pl.BlockSpec((1, D), lambda i: (0, 0)),
            pl.BlockSpec((1, D), lambda i: (0, 0)),
        ],
        out_specs=pl.BlockSpec((_BN, D), lambda i: (i, 0)),
        out_shape=jax.ShapeDtypeStruct((N, D), jnp.float32),
    )(h, agg, w1, b1.reshape(1, D), g.reshape(1, D), bt.reshape(1, D))


def _head_body(h_ref, w1_ref, b1_ref, w2_ref, b2_ref, o_ref):
    t = jnp.dot(h_ref[...], w1_ref[...], preferred_element_type=jnp.float32)
    t = jnp.maximum(t + b1_ref[...], 0.0)
    o_ref[...] = jnp.dot(t, w2_ref[...], preferred_element_type=jnp.float32) + b2_ref[...]


def _head(h, wh1, bh1, wh2, bh2):
    grid = (pl.cdiv(N, _BN),)
    return pl.pallas_call(
        _head_body,
        grid=grid,
        in_specs=[
            pl.BlockSpec((_BN, D), lambda i: (i, 0)),
            pl.BlockSpec((D, D), lambda i: (0, 0)),
            pl.BlockSpec((1, D), lambda i: (0, 0)),
            pl.BlockSpec((D, 1), lambda i: (0, 0)),
            pl.BlockSpec((1, 1), lambda i: (0, 0)),
        ],
        out_specs=pl.BlockSpec((_BN, 1), lambda i: (i, 0)),
        out_shape=jax.ShapeDtypeStruct((N, 1), jnp.float32),
    )(h, wh1, bh1.reshape(1, D), wh2, bh2.reshape(1, 1))


def _layernorm(t, g, bt):
    mu = jnp.mean(t, axis=-1, keepdims=True)
    var = jnp.mean((t - mu) ** 2, axis=-1, keepdims=True)
    return (t - mu) * jax.lax.rsqrt(var + 1e-5) * g + bt


def kernel(x, edge_index, edge_attr, params):
    src = edge_index[0].astype(jnp.int32)
    dst = edge_index[1].astype(jnp.int32)
    src3 = jnp.pad(src, (0, _EPAD - E))
    dst3 = jnp.pad(dst, (0, _EPAD - E), constant_values=N)

    ea_p = jnp.pad(edge_attr, ((0, _EPAD - E), (0, 0)))
    w_split = jnp.stack([params[f"We{i}"].reshape(ED, 2, 128).transpose(1, 0, 2)
                         for i in range(L)])
    b_split = jnp.stack([params[f"be{i}"].reshape(2, 1, 128) for i in range(L)])
    e_list = [jnp.einsum("ek,ckd->ced", ea_p, w_split[i],
                         preferred_element_type=jnp.float32) + b_split[i]
              for i in range(L)]

    h = x
    for i in range(L - 1):
        agg3 = _segment(h, e_list[i], src3, dst3)
        agg = jnp.concatenate([agg3[0, :N], agg3[1, :N]], axis=-1)
        t = (h + agg) @ params[f"W1{i}"] + params[f"b1{i}"]
        h = jnp.maximum(_layernorm(t, params[f"g{i}"], params[f"bt{i}"]), 0.0)

    i = L - 1
    agg3 = _segment(h, e_list[i], src3, dst3)
    h = _node_update(h, agg3, params[f"W1{i}"], params[f"b1{i}"],
                     params[f"g{i}"], params[f"bt{i}"])
    return _head(h, params["Wh1"], params["bh1"], params["Wh2"], params["bh2"])
